# baseline (speedup 1.0000x reference)
import functools

import jax
import jax.numpy as jnp
from jax import lax
from jax.experimental import pallas as pl
from jax.experimental.pallas import tpu as pltpu
from jax.experimental.pallas import tpu_sc as plsc

_N = 10000
_E = 320000
_D = 128
_CHUNK = 64
_NC = 2
_NS = 16
_NW = _NC * _NS
_RPW = 160
_NROWS = _RPW * _NW
_EPAD = _NROWS * _CHUNK
_NPAD = 10240
_BLK = 40
_NBUF = 4
_TILE_N = _NPAD // _NS


def _matmul_bias(x, w_t, b_row):
    def body(x_ref, w_ref, b_ref, o_ref):
        o_ref[...] = (
            jnp.dot(x_ref[...], w_ref[...], preferred_element_type=jnp.float32)
            + b_ref[...]
        )

    return pl.pallas_call(
        body,
        out_shape=jax.ShapeDtypeStruct((_N, _D), jnp.float32),
    )(x, w_t, b_row)


def _sc_segment_sum(h, s2d, r2d, zrows):
    mesh = plsc.VectorSubcoreMesh(core_axis_name="c", subcore_axis_name="s")

    @functools.partial(
        pl.kernel,
        mesh=mesh,
        out_type=jax.ShapeDtypeStruct((_NC * _NPAD, _D), jnp.float32),
        scratch_types=[
            pltpu.VMEM((_BLK, _CHUNK), jnp.int32),
            pltpu.VMEM((_BLK, _CHUNK), jnp.int32),
            pltpu.VMEM((_NBUF, _CHUNK, _D), jnp.float32),
            pltpu.VMEM_SHARED((_NPAD, _D), jnp.float32),
            pltpu.SemaphoreType.DMA((_NBUF,)),
            pltpu.SemaphoreType.DMA((_NBUF,)),
        ],
    )
    def k(h_hbm, s_hbm, r_hbm, z_hbm, out_hbm, s_v, r_v, rows_v, acc,
          gsem, ssem):
        c = lax.axis_index("c")
        sid = lax.axis_index("s")
        wid = sid * _NC + c

        pltpu.sync_copy(z_hbm, acc.at[pl.ds(sid * _TILE_N, _TILE_N)])

        plsc.subcore_barrier()

        def block(g, carry):
            blk = pl.multiple_of(wid * _RPW + g * _BLK, 8)
            pltpu.sync_copy(s_hbm.at[pl.ds(blk, _BLK)], s_v)
            pltpu.sync_copy(r_hbm.at[pl.ds(blk, _BLK)], r_v)

            for j in range(2):
                pltpu.async_copy(h_hbm.at[s_v.at[j]], rows_v.at[j],
                                 gsem.at[j])
            for j in range(_BLK):
                b = j % _NBUF
                if j >= 2:
                    pltpu.make_async_copy(
                        rows_v.at[(j - 2) % _NBUF],
                        acc.at[r_v.at[j - 2]],
                        ssem.at[(j - 2) % _NBUF]).wait()
                if j + 2 < _BLK:
                    pltpu.async_copy(h_hbm.at[s_v.at[j + 2]],
                                     rows_v.at[(j + 2) % _NBUF],
                                     gsem.at[(j + 2) % _NBUF])
                pltpu.make_async_copy(h_hbm.at[s_v.at[j]], rows_v.at[b],
                                      gsem.at[b]).wait()
                pltpu.async_copy(rows_v.at[b], acc.at[r_v.at[j]], ssem.at[b],
                                 add=True)
            for j in range(_BLK - 2, _BLK):
                b = j % _NBUF
                pltpu.make_async_copy(rows_v.at[b], acc.at[r_v.at[j]],
                                      ssem.at[b]).wait()
            return carry

        lax.fori_loop(0, _RPW // _BLK, block, 0)

        plsc.subcore_barrier()

        pltpu.sync_copy(
            acc.at[pl.ds(sid * _TILE_N, _TILE_N)],
            out_hbm.at[pl.ds(c * _NPAD + sid * _TILE_N, _TILE_N)],
        )

    return k(h, s2d, r2d, zrows)


def _combine_silu(parts):
    def body(p_ref, o_ref):
        t = p_ref[0, pl.ds(0, _N)] + p_ref[1, pl.ds(0, _N)]
        o_ref[...] = t * (1.0 / (1.0 + jnp.exp(-t)))

    return pl.pallas_call(
        body,
        out_shape=jax.ShapeDtypeStruct((_N, _D), jnp.float32),
    )(parts)


def kernel(x, adj, Wq_w, Wq_b, a_w, a_b, lin_w, lin_b):
    npad = _EPAD - _E
    pad_i = jnp.arange(npad, dtype=jnp.int32)
    s_pad = jnp.concatenate([adj[0], pad_i % _N])
    r_pad = jnp.concatenate([adj[1], _N + pad_i % (_NPAD - _N)])
    s2d = s_pad.reshape(_NROWS, _CHUNK)
    r2d = r_pad.reshape(_NROWS, _CHUNK)
    h = _matmul_bias(x, lin_w.T, lin_b.reshape(1, _D))
    zrows = jnp.zeros((_TILE_N, _D), jnp.float32)
    parts = _sc_segment_sum(h, s2d, r2d, zrows)
    return _combine_silu(parts.reshape(_NC, _NPAD, _D))


# speedup vs baseline: 1.1032x; 1.1032x over previous
import functools

import jax
import jax.numpy as jnp
import numpy as np
from jax import lax
from jax.experimental import pallas as pl
from jax.experimental.pallas import tpu as pltpu
from jax.experimental.pallas import tpu_sc as plsc

_N = 10000
_E = 320000
_D = 128
_CHUNK = 128
_NC = 2
_NS = 16
_NW = _NC * _NS
_RPW = 80
_NROWS = _E // _CHUNK
_MAIN = _NROWS - 4
_TAIL = 64
_NPAD = 10240
_BLK = 40
_TILE_N = _NPAD // _NS


def _matmul_bias(x, w, b_row):
    def body(x_ref, w_ref, b_ref, o_ref):
        o_ref[...] = lax.dot_general(
            x_ref[...], w_ref[...],
            dimension_numbers=(((1,), (1,)), ((), ())),
            preferred_element_type=jnp.float32,
        ) + b_ref[...]

    return pl.pallas_call(
        body,
        out_shape=jax.ShapeDtypeStruct((_N, _D), jnp.float32),
    )(x, w, b_row)


def _sc_segment_sum(h, adj3d, tail3d, zrows):
    mesh = plsc.VectorSubcoreMesh(core_axis_name="c", subcore_axis_name="s")

    @functools.partial(
        pl.kernel,
        mesh=mesh,
        out_type=jax.ShapeDtypeStruct((_NC * _NPAD, _D), jnp.float32),
        scratch_types=[
            pltpu.VMEM((_BLK, _CHUNK), jnp.int32),
            pltpu.VMEM((_BLK, _CHUNK), jnp.int32),
            pltpu.VMEM((_CHUNK, _D), jnp.float32),
            pltpu.VMEM((_CHUNK, _D), jnp.float32),
            pltpu.VMEM_SHARED((_NPAD, _D), jnp.float32),
            pltpu.SemaphoreType.DMA,
            pltpu.SemaphoreType.DMA,
        ],
    )
    def k(h_hbm, a_hbm, t_hbm, z_hbm, out_hbm, s_v, r_v, rows0, rows1, acc,
          sem0, sem1):
        c = lax.axis_index("c")
        sid = lax.axis_index("s")
        wid = sid * _NC + c

        pltpu.sync_copy(z_hbm, acc.at[pl.ds(sid * _TILE_N, _TILE_N)])

        plsc.subcore_barrier()

        rows = (rows0, rows1)
        sems = (sem0, sem1)

        def block(g, carry):
            last = wid == _NW - 1

            @pl.when(jnp.logical_not(last))
            def _():
                blk = pl.multiple_of(wid * _RPW + g * _BLK, 8)
                pltpu.sync_copy(a_hbm.at[0, pl.ds(blk, _BLK)], s_v)
                pltpu.sync_copy(a_hbm.at[1, pl.ds(blk, _BLK)], r_v)

            @pl.when(last & (g == 0))
            def _():
                pltpu.sync_copy(a_hbm.at[0, pl.ds(_MAIN - 16, 16)],
                                s_v.at[pl.ds(0, 16)])
                pltpu.sync_copy(a_hbm.at[1, pl.ds(_MAIN - 16, 16)],
                                r_v.at[pl.ds(0, 16)])
                pltpu.sync_copy(t_hbm.at[0, pl.ds(0, 24)],
                                s_v.at[pl.ds(16, 24)])
                pltpu.sync_copy(t_hbm.at[1, pl.ds(0, 24)],
                                r_v.at[pl.ds(16, 24)])

            @pl.when(last & (g == 1))
            def _():
                pltpu.sync_copy(t_hbm.at[0, pl.ds(24, 40)], s_v)
                pltpu.sync_copy(t_hbm.at[1, pl.ds(24, 40)], r_v)

            pltpu.async_copy(h_hbm.at[s_v.at[0]], rows0, sem0)
            pltpu.async_copy(h_hbm.at[s_v.at[1]], rows1, sem1)
            for j in range(_BLK):
                b = j % 2
                pltpu.make_async_copy(h_hbm.at[s_v.at[j]], rows[b],
                                      sems[b]).wait()
                pltpu.sync_copy(rows[b], acc.at[r_v.at[j]], add=True)
                if j + 2 < _BLK:
                    pltpu.async_copy(h_hbm.at[s_v.at[j + 2]], rows[b], sems[b])
            return carry

        lax.fori_loop(0, _RPW // _BLK, block, 0)

        plsc.subcore_barrier()

        pltpu.sync_copy(
            acc.at[pl.ds(sid * _TILE_N, _TILE_N)],
            out_hbm.at[pl.ds(c * _NPAD + sid * _TILE_N, _TILE_N)],
        )

    return k(h, adj3d, tail3d, zrows)


def _combine_silu(parts):
    def body(p_ref, o_ref):
        t = p_ref[0, pl.ds(0, _N)] + p_ref[1, pl.ds(0, _N)]
        o_ref[...] = t * (1.0 / (1.0 + jnp.exp(-t)))

    return pl.pallas_call(
        body,
        out_shape=jax.ShapeDtypeStruct((_N, _D), jnp.float32),
    )(parts)


_NPAD_E = _TAIL * _CHUNK - 4 * _CHUNK
_PAD_S = (np.arange(_NPAD_E, dtype=np.int32) % _N).reshape(60, _CHUNK)
_PAD_R = (_N + np.arange(_NPAD_E, dtype=np.int32) % (_NPAD - _N)).reshape(
    60, _CHUNK).astype(np.int32)


def kernel(x, adj, Wq_w, Wq_b, a_w, a_b, lin_w, lin_b):
    adj3d = adj.reshape(2, _NROWS, _CHUNK)
    tail_real = lax.slice(adj, (0, _MAIN * _CHUNK), (2, _E)).reshape(2, 4,
                                                                     _CHUNK)
    tail3d = jnp.concatenate(
        [tail_real,
         jnp.stack([jnp.asarray(_PAD_S), jnp.asarray(_PAD_R)])], axis=1)
    h = _matmul_bias(x, lin_w, lin_b.reshape(1, _D))
    zrows = jnp.zeros((_TILE_N, _D), jnp.float32)
    parts = _sc_segment_sum(h, adj3d, tail3d, zrows)
    return _combine_silu(parts.reshape(_NC, _NPAD, _D))


# speedup vs baseline: 1.1409x; 1.0342x over previous
import functools

import jax
import jax.numpy as jnp
import numpy as np
from jax import lax
from jax.experimental import pallas as pl
from jax.experimental.pallas import tpu as pltpu
from jax.experimental.pallas import tpu_sc as plsc

_N = 10000
_E = 320000
_D = 128
_CHUNK = 128
_NC = 2
_NS = 16
_NW = _NC * _NS
_RPW = 80
_NROWS = _E // _CHUNK
_MAIN = _NROWS - 4
_TAIL = 64
_NPAD = 10240
_BLK = 40
_TILE_N = _NPAD // _NS


def _matmul_bias(x, w, b_row):
    def body(x_ref, w_ref, b_ref, o_ref):
        o_ref[...] = lax.dot_general(
            x_ref[...], w_ref[...],
            dimension_numbers=(((1,), (1,)), ((), ())),
            preferred_element_type=jnp.float32,
        ) + b_ref[...]

    return pl.pallas_call(
        body,
        out_shape=jax.ShapeDtypeStruct((_N, _D), jnp.float32),
    )(x, w, b_row)


def _sc_segment_sum(h, adj3d, tail3d, zrows):
    mesh = plsc.VectorSubcoreMesh(core_axis_name="c", subcore_axis_name="s")

    @functools.partial(
        pl.kernel,
        mesh=mesh,
        out_type=jax.ShapeDtypeStruct((_NC * _NPAD, _D), jnp.float32),
        scratch_types=[
            pltpu.VMEM((_BLK * _CHUNK,), jnp.int32),
            pltpu.VMEM((_BLK * _CHUNK,), jnp.int32),
            pltpu.VMEM((_CHUNK, _D), jnp.float32),
            pltpu.VMEM((_CHUNK, _D), jnp.float32),
            pltpu.VMEM_SHARED((_NPAD, _D), jnp.float32),
            pltpu.SemaphoreType.DMA,
            pltpu.SemaphoreType.DMA,
        ],
    )
    def k(h_hbm, a_hbm, t_hbm, z_hbm, out_hbm, s_v, r_v, rows0, rows1, acc,
          sem0, sem1):
        c = lax.axis_index("c")
        sid = lax.axis_index("s")
        wid = sid * _NC + c

        pltpu.sync_copy(z_hbm, acc.at[pl.ds(sid * _TILE_N, _TILE_N)])

        plsc.subcore_barrier()

        rows = (rows0, rows1)
        sems = (sem0, sem1)

        def block(g, carry):
            last = wid == _NW - 1

            @pl.when(jnp.logical_not(last))
            def _():
                blk = pl.multiple_of((wid * _RPW + g * _BLK) * _CHUNK, 8)
                pltpu.sync_copy(a_hbm.at[0, pl.ds(blk, _BLK * _CHUNK)], s_v)
                pltpu.sync_copy(a_hbm.at[1, pl.ds(blk, _BLK * _CHUNK)], r_v)

            @pl.when(last & (g == 0))
            def _():
                m16 = (_MAIN - 16) * _CHUNK
                pltpu.sync_copy(a_hbm.at[0, pl.ds(m16, 16 * _CHUNK)],
                                s_v.at[pl.ds(0, 16 * _CHUNK)])
                pltpu.sync_copy(a_hbm.at[1, pl.ds(m16, 16 * _CHUNK)],
                                r_v.at[pl.ds(0, 16 * _CHUNK)])
                pltpu.sync_copy(t_hbm.at[0, pl.ds(0, 24 * _CHUNK)],
                                s_v.at[pl.ds(16 * _CHUNK, 24 * _CHUNK)])
                pltpu.sync_copy(t_hbm.at[1, pl.ds(0, 24 * _CHUNK)],
                                r_v.at[pl.ds(16 * _CHUNK, 24 * _CHUNK)])

            @pl.when(last & (g == 1))
            def _():
                pltpu.sync_copy(t_hbm.at[0, pl.ds(24 * _CHUNK, 40 * _CHUNK)],
                                s_v)
                pltpu.sync_copy(t_hbm.at[1, pl.ds(24 * _CHUNK, 40 * _CHUNK)],
                                r_v)

            pltpu.async_copy(h_hbm.at[s_v.at[pl.ds(0, _CHUNK)]], rows0, sem0)
            pltpu.async_copy(h_hbm.at[s_v.at[pl.ds(_CHUNK, _CHUNK)]], rows1, sem1)
            for j in range(_BLK):
                b = j % 2
                pltpu.make_async_copy(
                    h_hbm.at[s_v.at[pl.ds(j * _CHUNK, _CHUNK)]], rows[b],
                    sems[b]).wait()
                pltpu.sync_copy(
                    rows[b], acc.at[r_v.at[pl.ds(j * _CHUNK, _CHUNK)]],
                    add=True)
                if j + 2 < _BLK:
                    pltpu.async_copy(
                        h_hbm.at[s_v.at[pl.ds((j + 2) * _CHUNK, _CHUNK)]],
                        rows[b], sems[b])
            return carry

        lax.fori_loop(0, _RPW // _BLK, block, 0)

        plsc.subcore_barrier()

        pltpu.sync_copy(
            acc.at[pl.ds(sid * _TILE_N, _TILE_N)],
            out_hbm.at[pl.ds(c * _NPAD + sid * _TILE_N, _TILE_N)],
        )

    return k(h, adj3d, tail3d, zrows)


def _combine_silu(parts):
    def body(p_ref, o_ref):
        t = p_ref[0, pl.ds(0, _N)] + p_ref[1, pl.ds(0, _N)]
        o_ref[...] = t * (1.0 / (1.0 + jnp.exp(-t)))

    return pl.pallas_call(
        body,
        out_shape=jax.ShapeDtypeStruct((_N, _D), jnp.float32),
    )(parts)


_NPAD_E = _TAIL * _CHUNK - 4 * _CHUNK
_PAD_S = np.arange(_NPAD_E, dtype=np.int32) % _N
_PAD_R = (_N + np.arange(_NPAD_E, dtype=np.int32) % (_NPAD - _N)).astype(
    np.int32)


def kernel(x, adj, Wq_w, Wq_b, a_w, a_b, lin_w, lin_b):
    tail_real = lax.slice(adj, (0, _MAIN * _CHUNK), (2, _E))
    tail2d = jnp.concatenate(
        [tail_real,
         jnp.stack([jnp.asarray(_PAD_S), jnp.asarray(_PAD_R)])], axis=1)
    h = _matmul_bias(x, lin_w, lin_b.reshape(1, _D))
    zrows = jnp.asarray(np.zeros((_TILE_N, _D), np.float32))
    parts = _sc_segment_sum(h, adj, tail2d, zrows)
    return _combine_silu(parts.reshape(_NC, _NPAD, _D))


# speedup vs baseline: 1.1609x; 1.0175x over previous
import functools

import jax
import jax.numpy as jnp
import numpy as np
from jax import lax
from jax.experimental import pallas as pl
from jax.experimental.pallas import tpu as pltpu
from jax.experimental.pallas import tpu_sc as plsc

_N = 10000
_E = 320000
_D = 128
_CHUNK = 128
_NC = 2
_NS = 16
_NW = _NC * _NS
_RPW = 80
_NROWS = _E // _CHUNK
_MAIN = _NROWS - 4
_TAIL = 64
_NPAD = 10240
_BLK = 40
_TILE_N = _NPAD // _NS


def _matmul_bias(x, w, b_row):
    def body(x_ref, w_ref, b_ref, o_ref):
        o_ref[...] = lax.dot_general(
            x_ref[...], w_ref[...],
            dimension_numbers=(((1,), (1,)), ((), ())),
            preferred_element_type=jnp.float32,
        ) + b_ref[...]

    return pl.pallas_call(
        body,
        out_shape=jax.ShapeDtypeStruct((_N, _D), jnp.float32),
    )(x, w, b_row)


def _sc_segment_sum(h, adj3d, tail3d, zrows):
    mesh = plsc.VectorSubcoreMesh(core_axis_name="c", subcore_axis_name="s")

    @functools.partial(
        pl.kernel,
        mesh=mesh,
        out_type=jax.ShapeDtypeStruct((_NC * _NPAD, _D), jnp.float32),
        scratch_types=[
            pltpu.VMEM((_BLK * _CHUNK,), jnp.int32),
            pltpu.VMEM((_BLK * _CHUNK,), jnp.int32),
            pltpu.VMEM((_CHUNK, _D), jnp.float32),
            pltpu.VMEM((_CHUNK, _D), jnp.float32),
            pltpu.VMEM_SHARED((_NPAD, _D), jnp.float32),
            pltpu.SemaphoreType.DMA,
            pltpu.SemaphoreType.DMA,
            pltpu.SemaphoreType.DMA,
        ],
    )
    def k(h_hbm, a_hbm, t_hbm, z_hbm, out_hbm, s_v, r_v, rows0, rows1, acc,
          sem0, sem1, semz):
        c = lax.axis_index("c")
        sid = lax.axis_index("s")
        wid = sid * _NC + c

        pltpu.async_copy(z_hbm, acc.at[pl.ds(sid * _TILE_N, _TILE_N)], semz)

        rows = (rows0, rows1)
        sems = (sem0, sem1)
        last = wid == _NW - 1

        def stage(g):
            @pl.when(jnp.logical_not(last))
            def _():
                blk = pl.multiple_of((wid * _RPW + g * _BLK) * _CHUNK, 8)
                pltpu.sync_copy(a_hbm.at[0, pl.ds(blk, _BLK * _CHUNK)], s_v)
                pltpu.sync_copy(a_hbm.at[1, pl.ds(blk, _BLK * _CHUNK)], r_v)

            @pl.when(last & (g == 0))
            def _():
                m16 = (_MAIN - 16) * _CHUNK
                pltpu.sync_copy(a_hbm.at[0, pl.ds(m16, 16 * _CHUNK)],
                                s_v.at[pl.ds(0, 16 * _CHUNK)])
                pltpu.sync_copy(a_hbm.at[1, pl.ds(m16, 16 * _CHUNK)],
                                r_v.at[pl.ds(0, 16 * _CHUNK)])
                pltpu.sync_copy(t_hbm.at[0, pl.ds(0, 24 * _CHUNK)],
                                s_v.at[pl.ds(16 * _CHUNK, 24 * _CHUNK)])
                pltpu.sync_copy(t_hbm.at[1, pl.ds(0, 24 * _CHUNK)],
                                r_v.at[pl.ds(16 * _CHUNK, 24 * _CHUNK)])

            @pl.when(last & (g == 1))
            def _():
                pltpu.sync_copy(t_hbm.at[0, pl.ds(24 * _CHUNK, 40 * _CHUNK)],
                                s_v)
                pltpu.sync_copy(t_hbm.at[1, pl.ds(24 * _CHUNK, 40 * _CHUNK)],
                                r_v)

        def issue_head_gathers():
            pltpu.async_copy(h_hbm.at[s_v.at[pl.ds(0, _CHUNK)]], rows0, sem0)
            pltpu.async_copy(h_hbm.at[s_v.at[pl.ds(_CHUNK, _CHUNK)]], rows1,
                             sem1)

        stage(0)
        issue_head_gathers()
        pltpu.make_async_copy(
            z_hbm, acc.at[pl.ds(sid * _TILE_N, _TILE_N)], semz).wait()
        plsc.subcore_barrier()

        def block(g, carry):
            @pl.when(g > 0)
            def _():
                stage(g)
                issue_head_gathers()

            for j in range(_BLK):
                b = j % 2
                pltpu.make_async_copy(
                    h_hbm.at[s_v.at[pl.ds(j * _CHUNK, _CHUNK)]], rows[b],
                    sems[b]).wait()
                pltpu.sync_copy(
                    rows[b], acc.at[r_v.at[pl.ds(j * _CHUNK, _CHUNK)]],
                    add=True)
                if j + 2 < _BLK:
                    pltpu.async_copy(
                        h_hbm.at[s_v.at[pl.ds((j + 2) * _CHUNK, _CHUNK)]],
                        rows[b], sems[b])
            return carry

        lax.fori_loop(0, _RPW // _BLK, block, 0)

        plsc.subcore_barrier()

        pltpu.sync_copy(
            acc.at[pl.ds(sid * _TILE_N, _TILE_N)],
            out_hbm.at[pl.ds(c * _NPAD + sid * _TILE_N, _TILE_N)],
        )

    return k(h, adj3d, tail3d, zrows)


def _combine_silu(parts):
    def body(p_ref, o_ref):
        t = p_ref[0, pl.ds(0, _N)] + p_ref[1, pl.ds(0, _N)]
        o_ref[...] = t * (1.0 / (1.0 + jnp.exp(-t)))

    return pl.pallas_call(
        body,
        out_shape=jax.ShapeDtypeStruct((_N, _D), jnp.float32),
    )(parts)


_NPAD_E = _TAIL * _CHUNK - 4 * _CHUNK
_PAD_S = np.arange(_NPAD_E, dtype=np.int32) % _N
_PAD_R = (_N + np.arange(_NPAD_E, dtype=np.int32) % (_NPAD - _N)).astype(
    np.int32)


def kernel(x, adj, Wq_w, Wq_b, a_w, a_b, lin_w, lin_b):
    tail_real = lax.slice(adj, (0, _MAIN * _CHUNK), (2, _E))
    tail2d = jnp.concatenate(
        [tail_real,
         jnp.stack([jnp.asarray(_PAD_S), jnp.asarray(_PAD_R)])], axis=1)
    h = _matmul_bias(x, lin_w, lin_b.reshape(1, _D))
    zrows = jnp.asarray(np.zeros((_TILE_N, _D), np.float32))
    parts = _sc_segment_sum(h, adj, tail2d, zrows)
    return _combine_silu(parts.reshape(_NC, _NPAD, _D))
